# Initial kernel scaffold; baseline (speedup 1.0000x reference)
import jax
import jax.numpy as jnp
from jax.experimental import pallas as pl


def kernel(x, table):
    raise NotImplementedError("write your pallas kernel here")


# speedup vs baseline: 2.4811x; 2.4811x over previous
import jax
import jax.numpy as jnp
from jax import lax
from jax.experimental import pallas as pl
from jax.experimental.pallas import tpu as pltpu
from jax.experimental.pallas import tpu_sc as plsc

B, L, D, V = 4096, 50, 64, 100000
NC, NS = 2, 16
NW = NC * NS
S = B // NW


def _tec_body(x_hbm, table_hbm, out_hbm, idx_v, rows_v, t_v, sem):
    wid = lax.axis_index("s") * NC + lax.axis_index("c")
    base = wid * S
    pltpu.sync_copy(x_hbm.at[pl.ds(base, S)], idx_v)
    col = lax.iota(jnp.int32, 16) * L

    def sample_body(b, carry):
        pltpu.async_copy(table_hbm.at[idx_v.at[b]], rows_v, sem).wait()

        def jbody(j, c2):
            for k in range(4):
                data = rows_v[j, pl.ds(k * 16, 16)]
                tidx = col + (j + k * 16 * L)
                plsc.store_scatter(t_v, [tidx], data)
            return c2

        lax.fori_loop(0, L, jbody, 0, unroll=2)
        pltpu.sync_copy(t_v, out_hbm.at[base + b])
        return carry

    lax.fori_loop(0, S, sample_body, 0)


def kernel(x, table):
    x32 = x.astype(jnp.int32)
    mesh = plsc.VectorSubcoreMesh(core_axis_name="c", subcore_axis_name="s")
    f = pl.kernel(
        _tec_body,
        mesh=mesh,
        compiler_params=pltpu.CompilerParams(
            needs_layout_passes=False, use_tc_tiling_on_sc=False
        ),
        out_type=jax.ShapeDtypeStruct((B, D * L), jnp.float32),
        scratch_types=[
            pltpu.VMEM((S, L), jnp.int32),
            pltpu.VMEM((L, D), jnp.float32),
            pltpu.VMEM((D * L,), jnp.float32),
            pltpu.SemaphoreType.DMA,
        ],
    )
    out = f(x32, table)
    return out.reshape(B, D, L)


# speedup vs baseline: 3.1888x; 1.2852x over previous
import jax
import jax.numpy as jnp
from jax import lax
from jax.experimental import pallas as pl
from jax.experimental.pallas import tpu as pltpu
from jax.experimental.pallas import tpu_sc as plsc

B, L, D, V = 4096, 50, 64, 100000
NC, NS = 2, 16
NW = NC * NS
S = B // NW


def _tec_body(x_hbm, table_hbm, out_hbm, idx_v, rows_v, t_v, gsem, osem):
    wid = lax.axis_index("s") * NC + lax.axis_index("c")
    base = wid * S
    pltpu.sync_copy(x_hbm.at[pl.ds(base, S)], idx_v)
    col = lax.iota(jnp.int32, 16) * L

    def gather_issue(b, p):
        pltpu.async_copy(table_hbm.at[idx_v.at[b]], rows_v.at[p], gsem)

    def gather_wait(b, p):
        pltpu.make_async_copy(table_hbm.at[idx_v.at[b]], rows_v.at[p], gsem).wait()

    def out_issue(b, p):
        pltpu.async_copy(t_v.at[p], out_hbm.at[base + b], osem)

    def out_wait(b, p):
        pltpu.make_async_copy(t_v.at[p], out_hbm.at[base + b], osem).wait()

    gather_issue(0, 0)

    def sample_body(b, carry):
        p = lax.rem(b, 2)
        q = 1 - p
        gather_wait(b, p)
        nb = jnp.minimum(b + 1, S - 1)
        gather_issue(nb, q)

        @pl.when(b >= 2)
        def _():
            out_wait(b - 2, p)

        def jbody(j, c2):
            for k in range(4):
                data = rows_v[p, j, pl.ds(k * 16, 16)]
                tidx = col + (j + k * 16 * L)
                plsc.store_scatter(t_v.at[p], [tidx], data)
            return c2

        lax.fori_loop(0, L, jbody, 0, unroll=2)
        out_issue(b, p)
        return carry

    lax.fori_loop(0, S, sample_body, 0)
    gather_wait(S - 1, lax.rem(jnp.int32(S), 2))
    out_wait(S - 2, 0)
    out_wait(S - 1, 1)


def kernel(x, table):
    x32 = x.astype(jnp.int32)
    mesh = plsc.VectorSubcoreMesh(core_axis_name="c", subcore_axis_name="s")
    f = pl.kernel(
        _tec_body,
        mesh=mesh,
        compiler_params=pltpu.CompilerParams(
            needs_layout_passes=False, use_tc_tiling_on_sc=False
        ),
        out_type=jax.ShapeDtypeStruct((B, D * L), jnp.float32),
        scratch_types=[
            pltpu.VMEM((S, L), jnp.int32),
            pltpu.VMEM((2, L, D), jnp.float32),
            pltpu.VMEM((2, D * L), jnp.float32),
            pltpu.SemaphoreType.DMA,
            pltpu.SemaphoreType.DMA,
        ],
    )
    out = f(x32, table)
    return out.reshape(B, D, L)


# speedup vs baseline: 3.3272x; 1.0434x over previous
import jax
import jax.numpy as jnp
from jax import lax
from jax.experimental import pallas as pl
from jax.experimental.pallas import tpu as pltpu
from jax.experimental.pallas import tpu_sc as plsc

B, L, D, V = 4096, 50, 64, 100000
NC, NS = 2, 16
NW = NC * NS
S = B // NW
DEPTH = 4


def _tec_body(x_hbm, table_hbm, out_hbm, idx_v, rows_v, t_v, gsem, osem):
    wid = lax.axis_index("s") * NC + lax.axis_index("c")
    base = wid * S
    pltpu.sync_copy(x_hbm.at[pl.ds(base, S)], idx_v)
    col = lax.iota(jnp.int32, 16) * L

    def gather_issue(b, p):
        pltpu.async_copy(table_hbm.at[idx_v.at[b]], rows_v.at[p], gsem)

    def gather_wait(b, p):
        pltpu.make_async_copy(table_hbm.at[idx_v.at[b]], rows_v.at[p], gsem).wait()

    def out_issue(b, p):
        pltpu.async_copy(t_v.at[p], out_hbm.at[base + b], osem)

    def out_wait(b, p):
        pltpu.make_async_copy(t_v.at[p], out_hbm.at[base + b], osem).wait()

    for i in range(DEPTH):
        gather_issue(i, i)

    def sample_body(b, carry):
        p = lax.rem(b, DEPTH)
        gather_wait(b, p)

        @pl.when(b >= DEPTH)
        def _():
            out_wait(b - DEPTH, p)

        def jbody(j, c2):
            for k in range(4):
                data = rows_v[p, j, pl.ds(k * 16, 16)]
                tidx = col + (j + k * 16 * L)
                plsc.store_scatter(t_v.at[p], [tidx], data)
            return c2

        lax.fori_loop(0, L, jbody, 0, unroll=2)
        out_issue(b, p)
        nb = jnp.minimum(b + DEPTH, S - 1)
        gather_issue(nb, lax.rem(nb, DEPTH))
        return carry

    lax.fori_loop(0, S, sample_body, 0)
    for i in range(DEPTH):
        gather_wait(S - 1, lax.rem(jnp.int32(S - 1), DEPTH))
        out_wait(S - DEPTH + i, lax.rem(jnp.int32(S - DEPTH + i), DEPTH))


def kernel(x, table):
    x32 = x.astype(jnp.int32)
    mesh = plsc.VectorSubcoreMesh(core_axis_name="c", subcore_axis_name="s")
    f = pl.kernel(
        _tec_body,
        mesh=mesh,
        compiler_params=pltpu.CompilerParams(
            needs_layout_passes=False, use_tc_tiling_on_sc=False
        ),
        out_type=jax.ShapeDtypeStruct((B, D * L), jnp.float32),
        scratch_types=[
            pltpu.VMEM((S, L), jnp.int32),
            pltpu.VMEM((DEPTH, L, D), jnp.float32),
            pltpu.VMEM((DEPTH, D * L), jnp.float32),
            pltpu.SemaphoreType.DMA,
            pltpu.SemaphoreType.DMA,
        ],
    )
    out = f(x32, table)
    return out.reshape(B, D, L)
